# baseline (speedup 1.0000x reference)
import jax
import jax.numpy as jnp
import numpy as np
from jax.experimental import pallas as pl
from jax.experimental.pallas import tpu as pltpu

_B, _N, _K, _H = 4, 128, 30, 256
_NRBF, _NPOS, _DEPTH = 16, 16, 3
_FF = 4 * _H
_IB = 16


def _ln(x, s, b):
    mu = jnp.mean(x, axis=1, keepdims=True)
    xc = x - mu
    var = jnp.mean(xc * xc, axis=1, keepdims=True)
    return xc / jnp.sqrt(var + 1e-5) * s + b


def _fwd(xt_ref, xr_ref, v_ref, s_ref, a_ref, at_ref,
         wv_ref, bv_ref, we_ref, be_ref,
         w1_ref, b1_ref, w2_ref, b2_ref, w3_ref, b3_ref,
         n1s_ref, n1b_ref, wi_ref, bi_ref, wo_ref, bo_ref,
         n2s_ref, n2b_ref,
         wx_ref, bx_ref, ux_ref, bu_ref, tw_ref, tb_ref,
         h_out_ref, xo_ref,
         oh_scr, ef_scr, em_scr, dn_scr):
    for b in range(_B):
        _one_protein(b,
                     xt_ref, xr_ref, v_ref, s_ref, a_ref, at_ref,
                     wv_ref, bv_ref, we_ref, be_ref,
                     w1_ref, b1_ref, w2_ref, b2_ref, w3_ref, b3_ref,
                     n1s_ref, n1b_ref, wi_ref, bi_ref, wo_ref, bo_ref,
                     n2s_ref, n2b_ref,
                     wx_ref, bx_ref, ux_ref, bu_ref, tw_ref, tb_ref,
                     h_out_ref, xo_ref,
                     oh_scr, ef_scr, em_scr, dn_scr)


def _one_protein(b,
                 xt_ref, xr_ref, v_ref, s_ref, a_ref, at_ref,
                 wv_ref, bv_ref, we_ref, be_ref,
                 w1_ref, b1_ref, w2_ref, b2_ref, w3_ref, b3_ref,
                 n1s_ref, n1b_ref, wi_ref, bi_ref, wo_ref, bo_ref,
                 n2s_ref, n2b_ref,
                 wx_ref, bx_ref, ux_ref, bu_ref, tw_ref, tb_ref,
                 h_out_ref, xo_ref,
                 oh_scr, ef_scr, em_scr, dn_scr):
    f32 = jnp.float32
    bf16 = jnp.bfloat16

    vm = jnp.minimum(a_ref[b, :, 1:2], 1).astype(f32)
    vm_row = jnp.minimum(at_ref[b, 1:2, :], 1).astype(f32)
    amask = jnp.minimum(a_ref[b], 1).astype(f32)

    d2 = None
    for c in range(3):
        col = xt_ref[b, c, :, 1:2]
        row = xr_ref[b, c:c + 1, :]
        dc = col - row
        d2 = dc * dc if d2 is None else d2 + dc * dc
    dist = jnp.sqrt(d2 + 1e-6)
    m2d = vm * vm_row
    dadj = dist * m2d + (1.0 - m2d) * 1e4

    lane_i = jax.lax.broadcasted_iota(jnp.int32, (_N, _N), 1)
    row_i = jax.lax.broadcasted_iota(jnp.int32, (_N, 1), 0).astype(f32)
    mu_rbf = 2.0 + jax.lax.broadcasted_iota(jnp.int32, (1, _NRBF), 1).astype(f32) * (20.0 / (_NRBF - 1))
    sig = (22.0 - 2.0) / _NRBF
    freqs = jnp.exp(jax.lax.broadcasted_iota(jnp.int32, (1, _NPOS // 2), 1).astype(f32)
                    * (2.0 * (-np.log(10000.0) / _NPOS)))

    def tk_body(k, work):
        mn = jnp.min(work, axis=1, keepdims=True)
        ismin = work == mn
        idx = jnp.min(jnp.where(ismin, lane_i, _N), axis=1, keepdims=True)
        oh = (lane_i == idx).astype(f32)
        oh_scr[pl.ds(k * _N, _N), :] = oh
        dn_scr[pl.ds(k * _N, _N), :] = mn
        return jnp.where(lane_i == idx, 3e4, work)

    jax.lax.fori_loop(0, _K, tk_body, dadj)

    ang_i = row_i * freqs
    cos_i = jnp.cos(ang_i)
    sin_i = jnp.sin(ang_i)
    tab = jnp.concatenate([cos_i, sin_i, vm], axis=1)
    g = jnp.dot(oh_scr[:], tab)
    cos_e = g[:, 0:8]
    sin_e = g[:, 8:16]
    em = g[:, 16:17]
    em_scr[:] = em
    cnt = em[0:_N, :]
    for k in range(1, _K):
        cnt = cnt + em[k * _N:(k + 1) * _N, :]
    cos_it = jnp.concatenate([cos_i] * _K, axis=0)
    sin_it = jnp.concatenate([sin_i] * _K, axis=0)
    posc = cos_e * cos_it + sin_e * sin_it
    poss = sin_e * cos_it - cos_e * sin_it
    z = (dn_scr[:] - mu_rbf) / sig
    rbf = jnp.exp(-(z * z))
    ef_scr[:] = jnp.concatenate([posc, poss, rbf], axis=1)

    h = jnp.dot(v_ref[b], wv_ref[:]) + bv_ref[:]
    s_mat = s_ref[b]

    for l in range(_DEPTH):
        w1a = w1_ref[l, 0:_H, :]
        w1b = w1_ref[l, _H:2 * _H, :]
        w1c = w1_ref[l, 2 * _H:3 * _H, :]
        w1d = w1_ref[l, 3 * _H:4 * _H, :]
        u = jnp.dot(h, w1b) + jnp.dot(s_mat, w1c)
        ew = jnp.dot(we_ref[:], w1d)
        base = jnp.dot(h, w1a) + b1_ref[l] + jnp.dot(be_ref[:], w1d)
        w2 = w2_ref[l]
        b2 = b2_ref[l]

        nei = jnp.dot(oh_scr[:].astype(bf16), u.astype(bf16),
                      preferred_element_type=f32)
        ec = jnp.dot(ef_scr[:].astype(bf16), ew.astype(bf16),
                     preferred_element_type=f32)
        base_t = jnp.concatenate([base] * _K, axis=0)
        m1 = jax.nn.relu(nei + ec + base_t)
        m2m = jax.nn.relu(jnp.dot(m1.astype(bf16), w2.astype(bf16),
                                  preferred_element_type=f32) + b2) * em_scr[:]
        pooled = m2m[0:_N, :]
        for k in range(1, _K):
            pooled = pooled + m2m[k * _N:(k + 1) * _N, :]
        dh = jnp.dot(pooled, w3_ref[l]) * (1.0 / _K) + b3_ref[l] * (cnt * (1.0 / _K))
        h = _ln(h + dh, n1s_ref[l], n1b_ref[l])
        ff = jax.nn.relu(jnp.dot(h.astype(bf16), wi_ref[l].astype(bf16),
                                 preferred_element_type=f32) + bi_ref[l])
        h = _ln(h + jnp.dot(ff.astype(bf16), wo_ref[l].astype(bf16),
                            preferred_element_type=f32) + bo_ref[l],
                n2s_ref[l], n2b_ref[l])
        h = h * vm

    h_out_ref[b] = h

    p_m = jnp.dot(h, wx_ref[:]) + bx_ref[:]
    q_m = jnp.dot(h, ux_ref[:]) + bu_ref[:]
    denom = jnp.sum(vm, axis=0, keepdims=True) + 1e-6

    vm_t = jnp.concatenate([vm] * _IB, axis=0)
    xc_t = [jnp.concatenate([xt_ref[b, c]] * _IB, axis=0) for c in range(3)]
    q_t = jnp.concatenate([q_m] * _IB, axis=0)
    rsub = jax.lax.broadcasted_iota(jnp.int32, (_IB, _IB * _N), 0)
    rcol = jax.lax.broadcasted_iota(jnp.int32, (_IB, _IB * _N), 1)
    rones = ((rcol // _N) == rsub).astype(bf16)
    rones_t = ((rcol // _N) == rsub).astype(bf16).T
    tw_b = tw_ref[:].astype(bf16)

    for ib in range(_N // _IB):
        p_blk = p_m[ib * _IB:(ib + 1) * _IB, :].astype(bf16)
        rows = jax.nn.relu(
            jnp.dot(rones_t, p_blk, preferred_element_type=f32) + q_t)
        t = jnp.dot(rows.astype(bf16), tw_b,
                    preferred_element_type=f32) + tb_ref[:]
        tv = t * vm_t
        s1 = jnp.dot(rones, tv.astype(bf16), preferred_element_type=f32)
        ab = amask[ib * _IB:(ib + 1) * _IB, :]
        for c in range(3):
            s2 = jnp.dot(rones, (tv * xc_t[c]).astype(bf16),
                         preferred_element_type=f32)
            xc_blk = xt_ref[b, c, ib * _IB:(ib + 1) * _IB, :]
            fc = (xc_blk * s1 - s2) / denom
            fc = jnp.clip(fc, -20.0, 20.0)
            xo_ref[b, c, ib * _IB:(ib + 1) * _IB, :] = (xc_blk + fc) * ab


def kernel(X, V, S, A, params):
    p = params
    layers = p['layers']
    st = lambda name: jnp.stack([lp[name] for lp in layers])
    stb = lambda name: jnp.stack([lp[name] for lp in layers])[:, None, :]

    Xt = X.transpose(0, 3, 1, 2)
    Xr = X[:, :, 1, :].transpose(0, 2, 1)
    At = A.transpose(0, 2, 1)

    args = (
        Xt, Xr, V, S, A, At,
        p['Wv'], p['bv'][None, :], p['We'], p['be'][None, :],
        st('W1'), stb('b1'), st('W2'), stb('b2'), st('W3'), stb('b3'),
        stb('n1_s'), stb('n1_b'), st('Wi'), stb('bi'), st('Wo'), stb('bo'),
        stb('n2_s'), stb('n2_b'),
        p['Wx'], p['bx'][None, :], p['Ux'], p['bu'][None, :],
        p['Tw'], p['Tb'][None, :],
    )

    def bspec(a):
        shp = a.shape
        return pl.BlockSpec(shp, lambda: (0,) * len(shp))

    in_specs = [bspec(a) for a in args]

    h_out, xo = pl.pallas_call(
        _fwd,
        grid=(),
        in_specs=in_specs,
        out_specs=[
            pl.BlockSpec((_B, _N, _H), lambda: (0, 0, 0)),
            pl.BlockSpec((_B, 3, _N, 14), lambda: (0, 0, 0, 0)),
        ],
        out_shape=[
            jax.ShapeDtypeStruct((_B, _N, _H), jnp.float32),
            jax.ShapeDtypeStruct((_B, 3, _N, 14), jnp.float32),
        ],
        scratch_shapes=[
            pltpu.VMEM((_K * _N, _N), jnp.float32),
            pltpu.VMEM((_K * _N, 32), jnp.float32),
            pltpu.VMEM((_K * _N, 1), jnp.float32),
            pltpu.VMEM((_K * _N, 1), jnp.float32),
        ],
        compiler_params=pltpu.CompilerParams(
            vmem_limit_bytes=100 * 1024 * 1024,
        ),
    )(*args)

    return h_out, xo.transpose(0, 2, 3, 1)


# speedup vs baseline: 1.0733x; 1.0733x over previous
import jax
import jax.numpy as jnp
import numpy as np
from jax.experimental import pallas as pl
from jax.experimental.pallas import tpu as pltpu

_B, _N, _K, _H = 4, 128, 30, 256
_NRBF, _NPOS, _DEPTH = 16, 16, 3
_FF = 4 * _H
_IB = 16


def _ln(x, s, b):
    mu = jnp.mean(x, axis=1, keepdims=True)
    xc = x - mu
    var = jnp.mean(xc * xc, axis=1, keepdims=True)
    return xc / jnp.sqrt(var + 1e-5) * s + b


def _fwd(xt_ref, xr_ref, v_ref, s_ref, a_ref, at_ref,
         wv_ref, bv_ref, we_ref, be_ref,
         w1_ref, b1_ref, w2_ref, b2_ref, w3_ref, b3_ref,
         n1s_ref, n1b_ref, wi_ref, bi_ref, wo_ref, bo_ref,
         n2s_ref, n2b_ref,
         wx_ref, bx_ref, ux_ref, bu_ref, tw_ref, tb_ref,
         h_out_ref, xo_ref,
         oh_scr, ef_scr, em_scr, dn_scr, id_scr):
    _one_protein(0,
                 xt_ref, xr_ref, v_ref, s_ref, a_ref, at_ref,
                 wv_ref, bv_ref, we_ref, be_ref,
                 w1_ref, b1_ref, w2_ref, b2_ref, w3_ref, b3_ref,
                 n1s_ref, n1b_ref, wi_ref, bi_ref, wo_ref, bo_ref,
                 n2s_ref, n2b_ref,
                 wx_ref, bx_ref, ux_ref, bu_ref, tw_ref, tb_ref,
                 h_out_ref, xo_ref,
                 oh_scr, ef_scr, em_scr, dn_scr, id_scr)


def _one_protein(b,
                 xt_ref, xr_ref, v_ref, s_ref, a_ref, at_ref,
                 wv_ref, bv_ref, we_ref, be_ref,
                 w1_ref, b1_ref, w2_ref, b2_ref, w3_ref, b3_ref,
                 n1s_ref, n1b_ref, wi_ref, bi_ref, wo_ref, bo_ref,
                 n2s_ref, n2b_ref,
                 wx_ref, bx_ref, ux_ref, bu_ref, tw_ref, tb_ref,
                 h_out_ref, xo_ref,
                 oh_scr, ef_scr, em_scr, dn_scr, id_scr):
    f32 = jnp.float32
    bf16 = jnp.bfloat16

    vm = jnp.minimum(a_ref[b, :, 1:2], 1).astype(f32)
    vm_row = jnp.minimum(at_ref[b, 1:2, :], 1).astype(f32)
    amask = jnp.minimum(a_ref[b], 1).astype(f32)

    d2 = None
    for c in range(3):
        col = xt_ref[b, c, :, 1:2]
        row = xr_ref[b, c:c + 1, :]
        dc = col - row
        d2 = dc * dc if d2 is None else d2 + dc * dc
    dist = jnp.sqrt(d2 + 1e-6)
    m2d = vm * vm_row
    dadj = dist * m2d + (1.0 - m2d) * 1e4

    lane_i = jax.lax.broadcasted_iota(jnp.int32, (_N, _N), 1)
    row_i = jax.lax.broadcasted_iota(jnp.int32, (_N, 1), 0).astype(f32)
    mu_rbf = 2.0 + jax.lax.broadcasted_iota(jnp.int32, (1, _NRBF), 1).astype(f32) * (20.0 / (_NRBF - 1))
    sig = (22.0 - 2.0) / _NRBF
    freqs = jnp.exp(jax.lax.broadcasted_iota(jnp.int32, (1, _NPOS // 2), 1).astype(f32)
                    * (2.0 * (-np.log(10000.0) / _NPOS)))

    def tk_body(k, work):
        mn = jnp.min(work, axis=1, keepdims=True)
        ismin = work == mn
        idx = jnp.min(jnp.where(ismin, lane_i, _N), axis=1, keepdims=True)
        id_scr[pl.ds(k * _N, _N), :] = idx
        dn_scr[pl.ds(k * _N, _N), :] = mn
        return jnp.where(lane_i == idx, 3e4, work)

    jax.lax.fori_loop(0, _K, tk_body, dadj)

    lane_t = jax.lax.broadcasted_iota(jnp.int32, (_K * _N, _N), 1)
    oh_scr[:] = (lane_t == id_scr[:]).astype(bf16)
    ang_i = row_i * freqs
    cos_i = jnp.cos(ang_i)
    sin_i = jnp.sin(ang_i)
    tab = jnp.concatenate([cos_i, sin_i, vm], axis=1).astype(bf16)
    g = jnp.dot(oh_scr[:], tab, preferred_element_type=f32)
    cos_e = g[:, 0:8]
    sin_e = g[:, 8:16]
    em = g[:, 16:17]
    em_scr[:] = em
    cnt = em[0:_N, :]
    for k in range(1, _K):
        cnt = cnt + em[k * _N:(k + 1) * _N, :]
    cos_it = jnp.concatenate([cos_i] * _K, axis=0)
    sin_it = jnp.concatenate([sin_i] * _K, axis=0)
    posc = cos_e * cos_it + sin_e * sin_it
    poss = sin_e * cos_it - cos_e * sin_it
    z = (dn_scr[:] - mu_rbf) / sig
    rbf = jnp.exp(-(z * z))
    ef_scr[:] = jnp.concatenate([posc, poss, rbf], axis=1).astype(bf16)

    h = jnp.dot(v_ref[b], wv_ref[:]) + bv_ref[:]
    s_mat = s_ref[b]

    for l in range(_DEPTH):
        w1a = w1_ref[l, 0:_H, :]
        w1b = w1_ref[l, _H:2 * _H, :]
        w1c = w1_ref[l, 2 * _H:3 * _H, :]
        w1d = w1_ref[l, 3 * _H:4 * _H, :]
        u = jnp.dot(h, w1b) + jnp.dot(s_mat, w1c)
        ew = jnp.dot(we_ref[:], w1d)
        base = jnp.dot(h, w1a) + b1_ref[l] + jnp.dot(be_ref[:], w1d)
        w2 = w2_ref[l]
        b2 = b2_ref[l]

        nei = jnp.dot(oh_scr[:], u.astype(bf16),
                      preferred_element_type=f32)
        ec = jnp.dot(ef_scr[:], ew.astype(bf16),
                     preferred_element_type=f32)
        base_t = jnp.concatenate([base] * _K, axis=0)
        m1 = jax.nn.relu(nei + ec + base_t)
        m2m = jax.nn.relu(jnp.dot(m1.astype(bf16), w2.astype(bf16),
                                  preferred_element_type=f32) + b2) * em_scr[:]
        pooled = m2m[0:_N, :]
        for k in range(1, _K):
            pooled = pooled + m2m[k * _N:(k + 1) * _N, :]
        dh = jnp.dot(pooled, w3_ref[l]) * (1.0 / _K) + b3_ref[l] * (cnt * (1.0 / _K))
        h = _ln(h + dh, n1s_ref[l], n1b_ref[l])
        ff = jax.nn.relu(jnp.dot(h.astype(bf16), wi_ref[l].astype(bf16),
                                 preferred_element_type=f32) + bi_ref[l])
        h = _ln(h + jnp.dot(ff.astype(bf16), wo_ref[l].astype(bf16),
                            preferred_element_type=f32) + bo_ref[l],
                n2s_ref[l], n2b_ref[l])
        h = h * vm

    h_out_ref[b] = h

    p_m = jnp.dot(h, wx_ref[:]) + bx_ref[:]
    q_m = jnp.dot(h, ux_ref[:]) + bu_ref[:]
    denom = jnp.sum(vm, axis=0, keepdims=True) + 1e-6

    vm_t = jnp.concatenate([vm] * _IB, axis=0)
    xc_t = [jnp.concatenate([xt_ref[b, c]] * _IB, axis=0) for c in range(3)]
    q_t = jnp.concatenate([q_m] * _IB, axis=0)
    rsub = jax.lax.broadcasted_iota(jnp.int32, (_IB, _IB * _N), 0)
    rcol = jax.lax.broadcasted_iota(jnp.int32, (_IB, _IB * _N), 1)
    rones = ((rcol // _N) == rsub).astype(bf16)
    rones_t = ((rcol // _N) == rsub).astype(bf16).T
    tw_b = tw_ref[:].astype(bf16)

    for ib in range(_N // _IB):
        p_blk = p_m[ib * _IB:(ib + 1) * _IB, :].astype(bf16)
        rows = jax.nn.relu(
            jnp.dot(rones_t, p_blk, preferred_element_type=f32) + q_t)
        t = jnp.dot(rows.astype(bf16), tw_b,
                    preferred_element_type=f32) + tb_ref[:]
        tv = t * vm_t
        s1 = jnp.dot(rones, tv.astype(bf16), preferred_element_type=f32)
        ab = amask[ib * _IB:(ib + 1) * _IB, :]
        for c in range(3):
            s2 = jnp.dot(rones, (tv * xc_t[c]).astype(bf16),
                         preferred_element_type=f32)
            xc_blk = xt_ref[b, c, ib * _IB:(ib + 1) * _IB, :]
            fc = (xc_blk * s1 - s2) / denom
            fc = jnp.clip(fc, -20.0, 20.0)
            xo_ref[b, c, ib * _IB:(ib + 1) * _IB, :] = (xc_blk + fc) * ab


def kernel(X, V, S, A, params):
    p = params
    layers = p['layers']
    st = lambda name: jnp.stack([lp[name] for lp in layers])
    stb = lambda name: jnp.stack([lp[name] for lp in layers])[:, None, :]

    Xt = X.transpose(0, 3, 1, 2)
    Xr = X[:, :, 1, :].transpose(0, 2, 1)
    At = A.transpose(0, 2, 1)

    args = (
        Xt, Xr, V, S, A, At,
        p['Wv'], p['bv'][None, :], p['We'], p['be'][None, :],
        st('W1'), stb('b1'), st('W2'), stb('b2'), st('W3'), stb('b3'),
        stb('n1_s'), stb('n1_b'), st('Wi'), stb('bi'), st('Wo'), stb('bo'),
        stb('n2_s'), stb('n2_b'),
        p['Wx'], p['bx'][None, :], p['Ux'], p['bu'][None, :],
        p['Tw'], p['Tb'][None, :],
    )

    def bspec(a, batched):
        shp = a.shape
        if batched:
            blk = (1,) + shp[1:]
            return pl.BlockSpec(blk, lambda b: (b,) + (0,) * (len(shp) - 1))
        return pl.BlockSpec(shp, lambda b: (0,) * len(shp))

    in_specs = [bspec(a, i < 6) for i, a in enumerate(args)]

    h_out, xo = pl.pallas_call(
        _fwd,
        grid=(_B,),
        in_specs=in_specs,
        out_specs=[
            pl.BlockSpec((1, _N, _H), lambda b: (b, 0, 0)),
            pl.BlockSpec((1, 3, _N, 14), lambda b: (b, 0, 0, 0)),
        ],
        out_shape=[
            jax.ShapeDtypeStruct((_B, _N, _H), jnp.float32),
            jax.ShapeDtypeStruct((_B, 3, _N, 14), jnp.float32),
        ],
        scratch_shapes=[
            pltpu.VMEM((_K * _N, _N), jnp.bfloat16),
            pltpu.VMEM((_K * _N, 32), jnp.bfloat16),
            pltpu.VMEM((_K * _N, 1), jnp.float32),
            pltpu.VMEM((_K * _N, 1), jnp.float32),
            pltpu.VMEM((_K * _N, 1), jnp.int32),
        ],
        compiler_params=pltpu.CompilerParams(
            vmem_limit_bytes=100 * 1024 * 1024,
        ),
    )(*args)

    return h_out, xo.transpose(0, 2, 3, 1)


# speedup vs baseline: 1.1323x; 1.0550x over previous
import jax
import jax.numpy as jnp
import numpy as np
from jax.experimental import pallas as pl
from jax.experimental.pallas import tpu as pltpu

_B, _N, _K, _H = 4, 128, 30, 256
_NRBF, _NPOS, _DEPTH = 16, 16, 3
_FF = 4 * _H
_IB = 16


def _ln(x, s, b):
    mu = jnp.mean(x, axis=1, keepdims=True)
    xc = x - mu
    var = jnp.mean(xc * xc, axis=1, keepdims=True)
    return xc / jnp.sqrt(var + 1e-5) * s + b


def _fwd(xt_ref, xr_ref, v_ref, s_ref, a_ref, at_ref,
         wv_ref, bv_ref, we_ref, be_ref,
         w1_ref, b1_ref, w2_ref, b2_ref, w3_ref, b3_ref,
         n1s_ref, n1b_ref, wi_ref, bi_ref, wo_ref, bo_ref,
         n2s_ref, n2b_ref,
         wx_ref, bx_ref, ux_ref, bu_ref, tw_ref, tb_ref,
         h_out_ref, xo_ref,
         oh_scr, ef_scr, em_scr, dn_scr, id_scr):
    _one_protein(0,
                 xt_ref, xr_ref, v_ref, s_ref, a_ref, at_ref,
                 wv_ref, bv_ref, we_ref, be_ref,
                 w1_ref, b1_ref, w2_ref, b2_ref, w3_ref, b3_ref,
                 n1s_ref, n1b_ref, wi_ref, bi_ref, wo_ref, bo_ref,
                 n2s_ref, n2b_ref,
                 wx_ref, bx_ref, ux_ref, bu_ref, tw_ref, tb_ref,
                 h_out_ref, xo_ref,
                 oh_scr, ef_scr, em_scr, dn_scr, id_scr)


def _one_protein(b,
                 xt_ref, xr_ref, v_ref, s_ref, a_ref, at_ref,
                 wv_ref, bv_ref, we_ref, be_ref,
                 w1_ref, b1_ref, w2_ref, b2_ref, w3_ref, b3_ref,
                 n1s_ref, n1b_ref, wi_ref, bi_ref, wo_ref, bo_ref,
                 n2s_ref, n2b_ref,
                 wx_ref, bx_ref, ux_ref, bu_ref, tw_ref, tb_ref,
                 h_out_ref, xo_ref,
                 oh_scr, ef_scr, em_scr, dn_scr, id_scr):
    f32 = jnp.float32
    bf16 = jnp.bfloat16

    vm = jnp.minimum(a_ref[b, :, 1:2], 1).astype(f32)
    vm_row = jnp.minimum(at_ref[b, 1:2, :], 1).astype(f32)
    amask = jnp.minimum(a_ref[b], 1).astype(f32)

    d2 = None
    for c in range(3):
        col = xt_ref[b, c, :, 1:2]
        row = xr_ref[b, c:c + 1, :]
        dc = col - row
        d2 = dc * dc if d2 is None else d2 + dc * dc
    dist = jnp.sqrt(d2 + 1e-6)
    m2d = vm * vm_row
    dadj = dist * m2d + (1.0 - m2d) * 1e4

    lane_i = jax.lax.broadcasted_iota(jnp.int32, (_N, _N), 1)
    row_i = jax.lax.broadcasted_iota(jnp.int32, (_N, 1), 0).astype(f32)
    mu_rbf = 2.0 + jax.lax.broadcasted_iota(jnp.int32, (1, _NRBF), 1).astype(f32) * (20.0 / (_NRBF - 1))
    sig = (22.0 - 2.0) / _NRBF
    freqs = jnp.exp(jax.lax.broadcasted_iota(jnp.int32, (1, _NPOS // 2), 1).astype(f32)
                    * (2.0 * (-np.log(10000.0) / _NPOS)))

    work = dadj
    for k in range(_K):
        mn = jnp.min(work, axis=1, keepdims=True)
        ismin = work == mn
        idx = jnp.min(jnp.where(ismin, lane_i, _N), axis=1, keepdims=True)
        id_scr[k * _N:(k + 1) * _N, :] = idx
        dn_scr[k * _N:(k + 1) * _N, :] = mn
        work = jnp.where(lane_i == idx, 3e4, work)

    lane_t = jax.lax.broadcasted_iota(jnp.int32, (_K * _N, _N), 1)
    oh_scr[:] = (lane_t == id_scr[:]).astype(bf16)
    ang_i = row_i * freqs
    cos_i = jnp.cos(ang_i)
    sin_i = jnp.sin(ang_i)
    tab = jnp.concatenate([cos_i, sin_i, vm], axis=1).astype(bf16)
    g = jnp.dot(oh_scr[:], tab, preferred_element_type=f32)
    cos_e = g[:, 0:8]
    sin_e = g[:, 8:16]
    em = g[:, 16:17]
    em_scr[:] = em
    cnt = em[0:_N, :]
    for k in range(1, _K):
        cnt = cnt + em[k * _N:(k + 1) * _N, :]
    cos_it = jnp.concatenate([cos_i] * _K, axis=0)
    sin_it = jnp.concatenate([sin_i] * _K, axis=0)
    posc = cos_e * cos_it + sin_e * sin_it
    poss = sin_e * cos_it - cos_e * sin_it
    z = (dn_scr[:] - mu_rbf) / sig
    rbf = jnp.exp(-(z * z))
    ef_scr[:] = jnp.concatenate([posc, poss, rbf], axis=1).astype(bf16)

    h = jnp.dot(v_ref[b], wv_ref[:]) + bv_ref[:]
    s_mat = s_ref[b]

    for l in range(_DEPTH):
        w1a = w1_ref[l, 0:_H, :]
        w1b = w1_ref[l, _H:2 * _H, :]
        w1c = w1_ref[l, 2 * _H:3 * _H, :]
        w1d = w1_ref[l, 3 * _H:4 * _H, :]
        u = jnp.dot(h, w1b) + jnp.dot(s_mat, w1c)
        ew = jnp.dot(we_ref[:], w1d)
        base = jnp.dot(h, w1a) + b1_ref[l] + jnp.dot(be_ref[:], w1d)
        w2 = w2_ref[l]
        b2 = b2_ref[l]

        nei = jnp.dot(oh_scr[:], u.astype(bf16),
                      preferred_element_type=f32)
        ec = jnp.dot(ef_scr[:], ew.astype(bf16),
                     preferred_element_type=f32)
        base_t = jnp.concatenate([base] * _K, axis=0)
        m1 = jax.nn.relu(nei + ec + base_t)
        m2m = jax.nn.relu(jnp.dot(m1.astype(bf16), w2.astype(bf16),
                                  preferred_element_type=f32) + b2) * em_scr[:]
        pooled = m2m[0:_N, :]
        for k in range(1, _K):
            pooled = pooled + m2m[k * _N:(k + 1) * _N, :]
        dh = jnp.dot(pooled, w3_ref[l]) * (1.0 / _K) + b3_ref[l] * (cnt * (1.0 / _K))
        h = _ln(h + dh, n1s_ref[l], n1b_ref[l])
        ff = jax.nn.relu(jnp.dot(h.astype(bf16), wi_ref[l].astype(bf16),
                                 preferred_element_type=f32) + bi_ref[l])
        h = _ln(h + jnp.dot(ff.astype(bf16), wo_ref[l].astype(bf16),
                            preferred_element_type=f32) + bo_ref[l],
                n2s_ref[l], n2b_ref[l])
        h = h * vm

    h_out_ref[b] = h

    p_m = jnp.dot(h, wx_ref[:]) + bx_ref[:]
    q_m = jnp.dot(h, ux_ref[:]) + bu_ref[:]
    denom = jnp.sum(vm, axis=0, keepdims=True) + 1e-6

    vm_t = jnp.concatenate([vm] * _IB, axis=0)
    xc_t = [jnp.concatenate([xt_ref[b, c]] * _IB, axis=0) for c in range(3)]
    q_t = jnp.concatenate([q_m] * _IB, axis=0)
    rsub = jax.lax.broadcasted_iota(jnp.int32, (_IB, _IB * _N), 0)
    rcol = jax.lax.broadcasted_iota(jnp.int32, (_IB, _IB * _N), 1)
    rones = ((rcol // _N) == rsub).astype(bf16)
    rones_t = ((rcol // _N) == rsub).astype(bf16).T
    tw_b = tw_ref[:].astype(bf16)

    for ib in range(_N // _IB):
        p_blk = p_m[ib * _IB:(ib + 1) * _IB, :].astype(bf16)
        rows = jax.nn.relu(
            jnp.dot(rones_t, p_blk, preferred_element_type=f32) + q_t)
        t = jnp.dot(rows.astype(bf16), tw_b,
                    preferred_element_type=f32) + tb_ref[:]
        tv = t * vm_t
        s1 = jnp.dot(rones, tv.astype(bf16), preferred_element_type=f32)
        ab = amask[ib * _IB:(ib + 1) * _IB, :]
        for c in range(3):
            s2 = jnp.dot(rones, (tv * xc_t[c]).astype(bf16),
                         preferred_element_type=f32)
            xc_blk = xt_ref[b, c, ib * _IB:(ib + 1) * _IB, :]
            fc = (xc_blk * s1 - s2) / denom
            fc = jnp.clip(fc, -20.0, 20.0)
            xo_ref[b, c, ib * _IB:(ib + 1) * _IB, :] = (xc_blk + fc) * ab


def kernel(X, V, S, A, params):
    p = params
    layers = p['layers']
    st = lambda name: jnp.stack([lp[name] for lp in layers])
    stb = lambda name: jnp.stack([lp[name] for lp in layers])[:, None, :]

    Xt = X.transpose(0, 3, 1, 2)
    Xr = X[:, :, 1, :].transpose(0, 2, 1)
    At = A.transpose(0, 2, 1)

    args = (
        Xt, Xr, V, S, A, At,
        p['Wv'], p['bv'][None, :], p['We'], p['be'][None, :],
        st('W1'), stb('b1'), st('W2'), stb('b2'), st('W3'), stb('b3'),
        stb('n1_s'), stb('n1_b'), st('Wi'), stb('bi'), st('Wo'), stb('bo'),
        stb('n2_s'), stb('n2_b'),
        p['Wx'], p['bx'][None, :], p['Ux'], p['bu'][None, :],
        p['Tw'], p['Tb'][None, :],
    )

    def bspec(a, batched):
        shp = a.shape
        if batched:
            blk = (1,) + shp[1:]
            return pl.BlockSpec(blk, lambda b: (b,) + (0,) * (len(shp) - 1))
        return pl.BlockSpec(shp, lambda b: (0,) * len(shp))

    in_specs = [bspec(a, i < 6) for i, a in enumerate(args)]

    h_out, xo = pl.pallas_call(
        _fwd,
        grid=(_B,),
        in_specs=in_specs,
        out_specs=[
            pl.BlockSpec((1, _N, _H), lambda b: (b, 0, 0)),
            pl.BlockSpec((1, 3, _N, 14), lambda b: (b, 0, 0, 0)),
        ],
        out_shape=[
            jax.ShapeDtypeStruct((_B, _N, _H), jnp.float32),
            jax.ShapeDtypeStruct((_B, 3, _N, 14), jnp.float32),
        ],
        scratch_shapes=[
            pltpu.VMEM((_K * _N, _N), jnp.bfloat16),
            pltpu.VMEM((_K * _N, 32), jnp.bfloat16),
            pltpu.VMEM((_K * _N, 1), jnp.float32),
            pltpu.VMEM((_K * _N, 1), jnp.float32),
            pltpu.VMEM((_K * _N, 1), jnp.int32),
        ],
        compiler_params=pltpu.CompilerParams(
            vmem_limit_bytes=100 * 1024 * 1024,
        ),
    )(*args)

    return h_out, xo.transpose(0, 2, 3, 1)
